# Initial kernel scaffold; baseline (speedup 1.0000x reference)
import functools

import jax
import jax.numpy as jnp
from jax import lax
from jax.experimental import pallas as pl
from jax.experimental.pallas import tpu as pltpu
from jax.experimental.pallas import tpu_sc as plsc

EMBED = 32
_NC, _NS = 2, 16
_NW = _NC * _NS


def _make_gather(B):
    b_per_w = B // _NW
    C = 1024
    n_chunks = b_per_w // C
    assert b_per_w % C == 0 and B % _NW == 0

    mesh = plsc.VectorSubcoreMesh(core_axis_name="c", subcore_axis_name="s")

    @functools.partial(
        pl.kernel,
        mesh=mesh,
        out_type=jax.ShapeDtypeStruct((B, EMBED), jnp.float32),
        scratch_types=[
            pltpu.VMEM((C,), jnp.int32),
            pltpu.VMEM((C, EMBED), jnp.float32),
            pltpu.SemaphoreType.DMA,
        ],
    )
    def gather_k(table_hbm, idx_hbm, out_hbm, idx_v, rows_v, sem):
        wid = lax.axis_index("s") * _NC + lax.axis_index("c")
        base = wid * b_per_w

        def body(i, carry):
            off = base + i * C
            pltpu.sync_copy(idx_hbm.at[pl.ds(off, C)], idx_v)
            pltpu.async_copy(table_hbm.at[idx_v], rows_v, sem).wait()
            pltpu.sync_copy(rows_v, out_hbm.at[pl.ds(off, C)])
            return carry

        lax.fori_loop(0, n_chunks, body, 0)

    return gather_k


_gather = _make_gather(16384 * 26)


def kernel(input, weight):
    idx = input.reshape(-1)
    if idx.dtype != jnp.int32:
        idx = idx.astype(jnp.int32)
    out = _gather(weight, idx)
    return out.reshape(input.shape + (EMBED,))


# speedup vs baseline: 1.5464x; 1.5464x over previous
import functools

import jax
import jax.numpy as jnp
from jax import lax
from jax.experimental import pallas as pl
from jax.experimental.pallas import tpu as pltpu
from jax.experimental.pallas import tpu_sc as plsc

EMBED = 32
_NC, _NS = 2, 16
_NW = _NC * _NS


def _make_gather(B):
    b_per_w = B // _NW
    C = 1024
    n_chunks = b_per_w // C
    assert b_per_w % C == 0 and B % _NW == 0

    mesh = plsc.VectorSubcoreMesh(core_axis_name="c", subcore_axis_name="s")

    @functools.partial(
        pl.kernel,
        mesh=mesh,
        compiler_params=pltpu.CompilerParams(use_tc_tiling_on_sc=False),
        out_type=jax.ShapeDtypeStruct((B, EMBED), jnp.float32),
        scratch_types=[
            pltpu.VMEM((C,), jnp.int32),
            pltpu.VMEM((C, EMBED), jnp.float32),
            pltpu.SemaphoreType.DMA,
        ],
    )
    def gather_k(table_hbm, idx_hbm, out_hbm, idx_v, rows_v, sem):
        wid = lax.axis_index("s") * _NC + lax.axis_index("c")
        base = wid * b_per_w

        def body(i, carry):
            off = base + i * C
            pltpu.sync_copy(idx_hbm.at[pl.ds(off, C)], idx_v)
            pltpu.async_copy(table_hbm.at[idx_v], rows_v, sem).wait()
            pltpu.sync_copy(rows_v, out_hbm.at[pl.ds(off, C)])
            return carry

        lax.fori_loop(0, n_chunks, body, 0)

    return gather_k


_gather = _make_gather(16384 * 26)


def kernel(input, weight):
    idx = input.reshape(-1)
    if idx.dtype != jnp.int32:
        idx = idx.astype(jnp.int32)
    out = _gather(weight, idx)
    return out.reshape(input.shape + (EMBED,))


# speedup vs baseline: 1.5663x; 1.0129x over previous
import functools

import jax
import jax.numpy as jnp
from jax import lax
from jax.experimental import pallas as pl
from jax.experimental.pallas import tpu as pltpu
from jax.experimental.pallas import tpu_sc as plsc

EMBED = 32
_NC, _NS = 2, 16
_NW = _NC * _NS


def _make_gather(B):
    b_per_w = B // _NW
    C = 832
    NBUF = 4
    n_chunks = b_per_w // C
    n_blocks = n_chunks // NBUF
    assert b_per_w % C == 0 and n_chunks % NBUF == 0 and C % 8 == 0

    mesh = plsc.VectorSubcoreMesh(core_axis_name="c", subcore_axis_name="s")

    @functools.partial(
        pl.kernel,
        mesh=mesh,
        compiler_params=pltpu.CompilerParams(use_tc_tiling_on_sc=False),
        out_type=jax.ShapeDtypeStruct((B, EMBED), jnp.float32),
        scratch_types=[
            pltpu.VMEM((b_per_w,), jnp.int32),
            pltpu.VMEM((NBUF, C, EMBED), jnp.float32),
            pltpu.SemaphoreType.DMA((NBUF,)),
            pltpu.SemaphoreType.DMA((NBUF,)),
        ],
    )
    def gather_k(table_hbm, idx_hbm, out_hbm, idx_v, bufs, gsem, osem):
        wid = lax.axis_index("s") * _NC + lax.axis_index("c")
        base = wid * b_per_w

        pltpu.sync_copy(idx_hbm.at[pl.ds(base, b_per_w)], idx_v)

        def start_gather(i, j):
            pltpu.async_copy(
                table_hbm.at[idx_v.at[pl.ds(i * C, C)]], bufs.at[j], gsem.at[j])

        def wait_gather(i, j):
            pltpu.make_async_copy(
                table_hbm.at[idx_v.at[pl.ds(i * C, C)]], bufs.at[j],
                gsem.at[j]).wait()

        def start_out(i, j):
            pltpu.async_copy(
                bufs.at[j], out_hbm.at[pl.ds(base + i * C, C)], osem.at[j])

        def wait_out(i, j):
            pltpu.make_async_copy(
                bufs.at[j], out_hbm.at[pl.ds(base + i * C, C)],
                osem.at[j]).wait()

        for j in range(NBUF):
            start_gather(j, j)

        def block(blk, carry):
            for j in range(NBUF):
                i = blk * NBUF + j
                wait_gather(i, j)
                start_out(i, j)
            for j in range(NBUF):
                i = blk * NBUF + j
                wait_out(i, j)
                start_gather(i + NBUF, j)
            return carry

        lax.fori_loop(0, n_blocks - 1, block, 0)

        for j in range(NBUF):
            i = (n_blocks - 1) * NBUF + j
            wait_gather(i, j)
            start_out(i, j)
        for j in range(NBUF):
            i = (n_blocks - 1) * NBUF + j
            wait_out(i, j)

    return gather_k


_gather = _make_gather(16384 * 26)


def kernel(input, weight):
    idx = input.reshape(-1)
    if idx.dtype != jnp.int32:
        idx = idx.astype(jnp.int32)
    out = _gather(weight, idx)
    return out.reshape(input.shape + (EMBED,))
